# baseline (speedup 1.0000x reference)
import functools

import jax
import jax.numpy as jnp
from jax import lax
from jax.experimental import pallas as pl
from jax.experimental.pallas import tpu as pltpu
from jax.experimental.pallas import tpu_sc as plsc

N = 10000
E = 320000
D = 128
DH = D // 2
EPS = 1e-5

NC, NS = 2, 16
NW = NC * NS
CH = 128
NP = 10240
HR = 2 * NP

KA = 79
E_PAD_A = NW * KA * CH
KC = 157
E_PAD_C = NS * KC * CH

_MESH = plsc.VectorSubcoreMesh(
    core_axis_name="c", subcore_axis_name="s", num_cores=NC, num_subcores=NS)


def _degrees(src3, dst3, ones1, z1):
    BPT = HR // NS

    @functools.partial(
        pl.kernel,
        out_type=[jax.ShapeDtypeStruct((HR,), jnp.float32),
                  jax.ShapeDtypeStruct((HR,), jnp.float32)],
        mesh=_MESH,
        scratch_types=[
            pltpu.VMEM((KA, CH), jnp.int32),
            pltpu.VMEM((KA, CH), jnp.int32),
            pltpu.VMEM((CH,), jnp.float32),
            pltpu.VMEM_SHARED((HR,), jnp.float32),
            pltpu.SemaphoreType.DMA,
        ],
        compiler_params=pltpu.CompilerParams(use_tc_tiling_on_sc=False),
    )
    def body(src_h, dst_h, ones_h, z_h, out0, out1, srcv, dstv, onesv, hist, sem):
        c = lax.axis_index("c")
        s = lax.axis_index("s")
        wid = c * NS + s
        b0 = s * BPT
        d1 = pltpu.async_copy(src_h.at[wid], srcv, sem)
        d2 = pltpu.async_copy(dst_h.at[wid], dstv, sem)
        d3 = pltpu.async_copy(ones_h, onesv, sem)
        dz = pltpu.async_copy(z_h, hist.at[pl.ds(b0, BPT)], sem)
        d1.wait()
        d2.wait()
        d3.wait()
        dz.wait()
        plsc.subcore_barrier()
        pending = []
        for j in range(KA):
            pending.append(pltpu.async_copy(onesv, hist.at[srcv.at[j]], sem, add=True))
            pending.append(pltpu.async_copy(onesv, hist.at[dstv.at[j]], sem, add=True))
            if len(pending) >= 16:
                pending.pop(0).wait()
                pending.pop(0).wait()
        for dsc in pending:
            dsc.wait()
        plsc.subcore_barrier()

        @pl.when(c == 0)
        def _():
            pltpu.sync_copy(hist.at[pl.ds(b0, BPT)], out0.at[pl.ds(b0, BPT)])

        @pl.when(c == 1)
        def _():
            pltpu.sync_copy(hist.at[pl.ds(b0, BPT)], out1.at[pl.ds(b0, BPT)])

    return body(src3, dst3, ones1, z1)


def _aggregate(hf, src4, dst3, zblk):
    NB = 6
    GA = 5
    RZ = NP // NS
    WB = N // NS

    @functools.partial(
        pl.kernel,
        out_type=jax.ShapeDtypeStruct((NC, N, DH), jnp.float32),
        mesh=_MESH,
        scratch_types=[
            pltpu.VMEM((KC, CH), jnp.int32),
            pltpu.VMEM((KC, CH), jnp.int32),
            pltpu.VMEM((NB, CH, DH), jnp.float32),
            pltpu.VMEM_SHARED((NP, DH), jnp.float32),
            pltpu.SemaphoreType.DMA,
            pltpu.SemaphoreType.DMA,
        ],
        compiler_params=pltpu.CompilerParams(use_tc_tiling_on_sc=False),
    )
    def body(h_h, src_h, dst_h, z_h, out_h, srcv, dstv, bufs, accum, gsem, ssem):
        c = lax.axis_index("c")
        s = lax.axis_index("s")
        hc = h_h
        di = pltpu.async_copy(src_h.at[c, s], srcv, gsem)
        dj = pltpu.async_copy(dst_h.at[s], dstv, gsem)
        dz = pltpu.async_copy(z_h, accum.at[pl.ds(s * RZ, RZ)], ssem)
        di.wait()
        dj.wait()
        dz.wait()
        plsc.subcore_barrier()
        gd = {}
        sd = {}
        for j in range(GA):
            gd[j] = pltpu.async_copy(hc.at[srcv.at[j]], bufs.at[j % NB], gsem)
        for j in range(KC):
            gd[j].wait()
            sd[j] = pltpu.async_copy(bufs.at[j % NB], accum.at[dstv.at[j]], ssem,
                                     add=True)
            nxt = j + GA
            if nxt < KC:
                prev = nxt - NB
                if prev >= 0:
                    sd[prev].wait()
                gd[nxt] = pltpu.async_copy(hc.at[srcv.at[nxt]], bufs.at[nxt % NB],
                                           gsem)
        for j in range(KC - NB, KC):
            sd[j].wait()
        plsc.subcore_barrier()
        pltpu.sync_copy(accum.at[pl.ds(s * WB, WB)], out_h.at[c, pl.ds(s * WB, WB)])

    return body(hf, src4, dst3, zblk)


_R = 1000


def _matmul0(x, W0, b0r):
    def body(x_r, w_r, b_r, o_r):
        o_r[...] = jnp.dot(x_r[...], w_r[...],
                           preferred_element_type=jnp.float32) + b_r[...]

    return pl.pallas_call(
        body,
        grid=(N // _R,),
        in_specs=[
            pl.BlockSpec((_R, D), lambda i: (i, 0)),
            pl.BlockSpec((D, D), lambda i: (0, 0)),
            pl.BlockSpec((1, D), lambda i: (0, 0)),
        ],
        out_specs=pl.BlockSpec((_R, D), lambda i: (i, 0)),
        out_shape=jax.ShapeDtypeStruct((N, D), jnp.float32),
    )(x, W0, b0r)


def _scale0(m, dego):
    def body(m_r, dg_r, o_r):
        a = lax.rsqrt(jnp.maximum(dg_r[...], 1.0))
        o_r[...] = m_r[...] * a

    return pl.pallas_call(
        body,
        grid=(N // _R,),
        in_specs=[
            pl.BlockSpec((_R, D), lambda i: (i, 0)),
            pl.BlockSpec((_R, 1), lambda i: (i, 0)),
        ],
        out_specs=pl.BlockSpec((_R, D), lambda i: (i, 0)),
        out_shape=jax.ShapeDtypeStruct((N, D), jnp.float32),
    )(m, dego)


def _dense1(acc, degi, dego, g_r_, lb_r_, W1, b1r):
    def body(ac_r, di_r, do_r, g_r, b_r, w_r, bias_r, o_r):
        t = jnp.concatenate([ac_r[0], ac_r[1]], axis=-1)
        cc = lax.rsqrt(jnp.maximum(di_r[...], 1.0))
        t = t * cc
        mu = jnp.mean(t, axis=-1, keepdims=True)
        tc0 = t - mu
        var = jnp.mean(tc0 * tc0, axis=-1, keepdims=True)
        t = tc0 * lax.rsqrt(var + EPS) * g_r[...] + b_r[...]
        t = jnp.maximum(t, 0.0)
        a = lax.rsqrt(jnp.maximum(do_r[...], 1.0))
        o_r[...] = (jnp.dot(t, w_r[...],
                            preferred_element_type=jnp.float32) + bias_r[...]) * a

    return pl.pallas_call(
        body,
        grid=(N // _R,),
        in_specs=[
            pl.BlockSpec((NC, _R, DH), lambda i: (0, i, 0)),
            pl.BlockSpec((_R, 1), lambda i: (i, 0)),
            pl.BlockSpec((_R, 1), lambda i: (i, 0)),
            pl.BlockSpec((1, D), lambda i: (0, 0)),
            pl.BlockSpec((1, D), lambda i: (0, 0)),
            pl.BlockSpec((D, D), lambda i: (0, 0)),
            pl.BlockSpec((1, D), lambda i: (0, 0)),
        ],
        out_specs=pl.BlockSpec((_R, D), lambda i: (i, 0)),
        out_shape=jax.ShapeDtypeStruct((N, D), jnp.float32),
    )(acc, degi, dego, g_r_, lb_r_, W1, b1r)


def _dense2(acc, degi, g_r_, lb_r_):
    def body(ac_r, di_r, g_r, b_r, o_r):
        t = jnp.concatenate([ac_r[0], ac_r[1]], axis=-1)
        cc = lax.rsqrt(jnp.maximum(di_r[...], 1.0))
        t = t * cc
        mu = jnp.mean(t, axis=-1, keepdims=True)
        tc0 = t - mu
        var = jnp.mean(tc0 * tc0, axis=-1, keepdims=True)
        t = tc0 * lax.rsqrt(var + EPS) * g_r[...] + b_r[...]
        o_r[...] = jnp.maximum(t, 0.0)

    return pl.pallas_call(
        body,
        grid=(N // _R,),
        in_specs=[
            pl.BlockSpec((NC, _R, DH), lambda i: (0, i, 0)),
            pl.BlockSpec((_R, 1), lambda i: (i, 0)),
            pl.BlockSpec((1, D), lambda i: (0, 0)),
            pl.BlockSpec((1, D), lambda i: (0, 0)),
        ],
        out_specs=pl.BlockSpec((_R, D), lambda i: (i, 0)),
        out_shape=jax.ShapeDtypeStruct((N, D), jnp.float32),
    )(acc, degi, g_r_, lb_r_)


def kernel(x, edge_index, W0, b0, W1, b1, ln_g0, ln_b0, ln_g1, ln_b1):
    src = edge_index[0]
    dst = edge_index[1]
    padA = E_PAD_A - E
    padC = E_PAD_C - E
    srcA = jnp.concatenate(
        [src, jnp.full((padA,), NP - 1, jnp.int32)]).reshape(NW, KA, CH)
    dstA = jnp.concatenate(
        [dst + NP, jnp.full((padA,), HR - 1, jnp.int32)]).reshape(NW, KA, CH)
    srcC = jnp.concatenate(
        [src, jnp.zeros((padC,), jnp.int32)]).reshape(NS, KC, CH)
    srcF = (2 * srcC)[None] + jnp.arange(NC, dtype=jnp.int32).reshape(NC, 1, 1, 1)
    dstC = jnp.concatenate(
        [dst, jnp.full((padC,), NP - 1, jnp.int32)]).reshape(NS, KC, CH)
    ones1 = jnp.ones((CH,), jnp.float32)
    z1 = jnp.zeros((HR // NS,), jnp.float32)
    zblk = jnp.zeros((NP // NS, DH), jnp.float32)

    deg0, deg1 = _degrees(srcA, dstA, ones1, z1)
    deg = deg0 + deg1
    dego = deg[:N, None]
    degi = deg[NP:NP + N, None]

    b0r = b0.reshape(1, D)
    b1r = b1.reshape(1, D)
    g0r = ln_g0.reshape(1, D)
    lb0r = ln_b0.reshape(1, D)
    g1r = ln_g1.reshape(1, D)
    lb1r = ln_b1.reshape(1, D)

    m0 = _matmul0(x, W0, b0r)
    h0 = _scale0(m0, dego)
    acc0 = _aggregate(h0.reshape(2 * N, DH), srcF, dstC, zblk)
    h1 = _dense1(acc0, degi, dego, g0r, lb0r, W1, b1r)
    acc1 = _aggregate(h1.reshape(2 * N, DH), srcF, dstC, zblk)
    return _dense2(acc1, degi, g1r, lb1r)


# speedup vs baseline: 1.0333x; 1.0333x over previous
import functools

import jax
import jax.numpy as jnp
from jax import lax
from jax.experimental import pallas as pl
from jax.experimental.pallas import tpu as pltpu
from jax.experimental.pallas import tpu_sc as plsc

N = 10000
E = 320000
D = 128
DH = D // 2
EPS = 1e-5

NC, NS = 2, 16
NW = NC * NS
CH = 128
NP = 10240
HR = 2 * NP

KA = 79
E_PAD_A = NW * KA * CH
KC = 157
E_PAD_C = NS * KC * CH

_MESH = plsc.VectorSubcoreMesh(
    core_axis_name="c", subcore_axis_name="s", num_cores=NC, num_subcores=NS)


def _degrees(src3, dst3, ones1, z1):
    BPT = HR // NS

    @functools.partial(
        pl.kernel,
        out_type=[jax.ShapeDtypeStruct((HR,), jnp.float32),
                  jax.ShapeDtypeStruct((HR,), jnp.float32)],
        mesh=_MESH,
        scratch_types=[
            pltpu.VMEM((KA, CH), jnp.int32),
            pltpu.VMEM((KA, CH), jnp.int32),
            pltpu.VMEM((CH,), jnp.float32),
            pltpu.VMEM_SHARED((HR,), jnp.float32),
            pltpu.SemaphoreType.DMA,
        ],
        compiler_params=pltpu.CompilerParams(use_tc_tiling_on_sc=False),
    )
    def body(src_h, dst_h, ones_h, z_h, out0, out1, srcv, dstv, onesv, hist, sem):
        c = lax.axis_index("c")
        s = lax.axis_index("s")
        wid = c * NS + s
        b0 = s * BPT
        d1 = pltpu.async_copy(src_h.at[wid], srcv, sem)
        d2 = pltpu.async_copy(dst_h.at[wid], dstv, sem)
        d3 = pltpu.async_copy(ones_h, onesv, sem)
        dz = pltpu.async_copy(z_h, hist.at[pl.ds(b0, BPT)], sem)
        d1.wait()
        d2.wait()
        d3.wait()
        dz.wait()
        plsc.subcore_barrier()
        pending = []
        for j in range(KA):
            pending.append(pltpu.async_copy(onesv, hist.at[srcv.at[j]], sem, add=True))
            pending.append(pltpu.async_copy(onesv, hist.at[dstv.at[j]], sem, add=True))
            if len(pending) >= 16:
                pending.pop(0).wait()
                pending.pop(0).wait()
        for dsc in pending:
            dsc.wait()
        plsc.subcore_barrier()

        @pl.when(c == 0)
        def _():
            pltpu.sync_copy(hist.at[pl.ds(b0, BPT)], out0.at[pl.ds(b0, BPT)])

        @pl.when(c == 1)
        def _():
            pltpu.sync_copy(hist.at[pl.ds(b0, BPT)], out1.at[pl.ds(b0, BPT)])

    return body(src3, dst3, ones1, z1)


def _aggregate(h2, src3, dst3, zblk):
    NB = 6
    GA = 5
    RZ = NP // NS
    WB = N // NS

    @functools.partial(
        pl.kernel,
        out_type=jax.ShapeDtypeStruct((NC, N, DH), jnp.float32),
        mesh=_MESH,
        scratch_types=[
            pltpu.VMEM((KC, CH), jnp.int32),
            pltpu.VMEM((KC, CH), jnp.int32),
            pltpu.VMEM((NB, CH, DH), jnp.float32),
            pltpu.VMEM_SHARED((NP, DH), jnp.float32),
            pltpu.SemaphoreType.DMA,
            pltpu.SemaphoreType.DMA,
        ],
        compiler_params=pltpu.CompilerParams(use_tc_tiling_on_sc=False),
    )
    def body(h_h, src_h, dst_h, z_h, out_h, srcv, dstv, bufs, accum, gsem, ssem):
        c = lax.axis_index("c")
        s = lax.axis_index("s")
        hc = h_h.at[c]
        di = pltpu.async_copy(src_h.at[s], srcv, gsem)
        dj = pltpu.async_copy(dst_h.at[s], dstv, gsem)
        dz = pltpu.async_copy(z_h, accum.at[pl.ds(s * RZ, RZ)], ssem)
        di.wait()
        dj.wait()
        dz.wait()
        plsc.subcore_barrier()
        gd = {}
        sd = {}
        for j in range(GA):
            gd[j] = pltpu.async_copy(hc.at[srcv.at[j]], bufs.at[j % NB], gsem)
        for j in range(KC):
            gd[j].wait()
            sd[j] = pltpu.async_copy(bufs.at[j % NB], accum.at[dstv.at[j]], ssem,
                                     add=True)
            nxt = j + GA
            if nxt < KC:
                prev = nxt - NB
                if prev >= 0:
                    sd[prev].wait()
                gd[nxt] = pltpu.async_copy(hc.at[srcv.at[nxt]], bufs.at[nxt % NB],
                                           gsem)
        for j in range(KC - NB, KC):
            sd[j].wait()
        plsc.subcore_barrier()
        pltpu.sync_copy(accum.at[pl.ds(s * WB, WB)], out_h.at[c, pl.ds(s * WB, WB)])

    return body(h2, src3, dst3, zblk)


_R = 1000


def _matmul0(x, W0, b0r):
    def body(x_r, w_r, b_r, o_r):
        o_r[...] = jnp.dot(x_r[...], w_r[...],
                           preferred_element_type=jnp.float32) + b_r[...]

    return pl.pallas_call(
        body,
        grid=(N // _R,),
        in_specs=[
            pl.BlockSpec((_R, D), lambda i: (i, 0)),
            pl.BlockSpec((D, D), lambda i: (0, 0)),
            pl.BlockSpec((1, D), lambda i: (0, 0)),
        ],
        out_specs=pl.BlockSpec((_R, D), lambda i: (i, 0)),
        out_shape=jax.ShapeDtypeStruct((N, D), jnp.float32),
    )(x, W0, b0r)


def _scale0(m, dego):
    def body(m_r, dg_r, o_r):
        a = lax.rsqrt(jnp.maximum(dg_r[...], 1.0))
        t = m_r[...] * a
        o_r[0] = t[:, :DH]
        o_r[1] = t[:, DH:]

    return pl.pallas_call(
        body,
        grid=(N // _R,),
        in_specs=[
            pl.BlockSpec((_R, D), lambda i: (i, 0)),
            pl.BlockSpec((_R, 1), lambda i: (i, 0)),
        ],
        out_specs=pl.BlockSpec((NC, _R, DH), lambda i: (0, i, 0)),
        out_shape=jax.ShapeDtypeStruct((NC, N, DH), jnp.float32),
    )(m, dego)


def _dense1(acc, degi, dego, g_r_, lb_r_, W1, b1r):
    def body(ac_r, di_r, do_r, g_r, b_r, w_r, bias_r, o_r):
        t = jnp.concatenate([ac_r[0], ac_r[1]], axis=-1)
        cc = lax.rsqrt(jnp.maximum(di_r[...], 1.0))
        t = t * cc
        mu = jnp.mean(t, axis=-1, keepdims=True)
        tc0 = t - mu
        var = jnp.mean(tc0 * tc0, axis=-1, keepdims=True)
        t = tc0 * lax.rsqrt(var + EPS) * g_r[...] + b_r[...]
        t = jnp.maximum(t, 0.0)
        a = lax.rsqrt(jnp.maximum(do_r[...], 1.0))
        t = (jnp.dot(t, w_r[...],
                     preferred_element_type=jnp.float32) + bias_r[...]) * a
        o_r[0] = t[:, :DH]
        o_r[1] = t[:, DH:]

    return pl.pallas_call(
        body,
        grid=(N // _R,),
        in_specs=[
            pl.BlockSpec((NC, _R, DH), lambda i: (0, i, 0)),
            pl.BlockSpec((_R, 1), lambda i: (i, 0)),
            pl.BlockSpec((_R, 1), lambda i: (i, 0)),
            pl.BlockSpec((1, D), lambda i: (0, 0)),
            pl.BlockSpec((1, D), lambda i: (0, 0)),
            pl.BlockSpec((D, D), lambda i: (0, 0)),
            pl.BlockSpec((1, D), lambda i: (0, 0)),
        ],
        out_specs=pl.BlockSpec((NC, _R, DH), lambda i: (0, i, 0)),
        out_shape=jax.ShapeDtypeStruct((NC, N, DH), jnp.float32),
    )(acc, degi, dego, g_r_, lb_r_, W1, b1r)


def _dense2(acc, degi, g_r_, lb_r_):
    def body(ac_r, di_r, g_r, b_r, o_r):
        t = jnp.concatenate([ac_r[0], ac_r[1]], axis=-1)
        cc = lax.rsqrt(jnp.maximum(di_r[...], 1.0))
        t = t * cc
        mu = jnp.mean(t, axis=-1, keepdims=True)
        tc0 = t - mu
        var = jnp.mean(tc0 * tc0, axis=-1, keepdims=True)
        t = tc0 * lax.rsqrt(var + EPS) * g_r[...] + b_r[...]
        o_r[...] = jnp.maximum(t, 0.0)

    return pl.pallas_call(
        body,
        grid=(N // _R,),
        in_specs=[
            pl.BlockSpec((NC, _R, DH), lambda i: (0, i, 0)),
            pl.BlockSpec((_R, 1), lambda i: (i, 0)),
            pl.BlockSpec((1, D), lambda i: (0, 0)),
            pl.BlockSpec((1, D), lambda i: (0, 0)),
        ],
        out_specs=pl.BlockSpec((_R, D), lambda i: (i, 0)),
        out_shape=jax.ShapeDtypeStruct((N, D), jnp.float32),
    )(acc, degi, g_r_, lb_r_)


def kernel(x, edge_index, W0, b0, W1, b1, ln_g0, ln_b0, ln_g1, ln_b1):
    src = edge_index[0]
    dst = edge_index[1]
    padA = E_PAD_A - E
    padC = E_PAD_C - E
    srcA = jnp.concatenate(
        [src, jnp.full((padA,), NP - 1, jnp.int32)]).reshape(NW, KA, CH)
    dstA = jnp.concatenate(
        [dst + NP, jnp.full((padA,), HR - 1, jnp.int32)]).reshape(NW, KA, CH)
    srcC = jnp.concatenate(
        [src, jnp.zeros((padC,), jnp.int32)]).reshape(NS, KC, CH)
    dstC = jnp.concatenate(
        [dst, jnp.full((padC,), NP - 1, jnp.int32)]).reshape(NS, KC, CH)
    ones1 = jnp.ones((CH,), jnp.float32)
    z1 = jnp.zeros((HR // NS,), jnp.float32)
    zblk = jnp.zeros((NP // NS, DH), jnp.float32)

    deg0, deg1 = _degrees(srcA, dstA, ones1, z1)
    deg = deg0 + deg1
    dego = deg[:N, None]
    degi = deg[NP:NP + N, None]

    b0r = b0.reshape(1, D)
    b1r = b1.reshape(1, D)
    g0r = ln_g0.reshape(1, D)
    lb0r = ln_b0.reshape(1, D)
    g1r = ln_g1.reshape(1, D)
    lb1r = ln_b1.reshape(1, D)

    m0 = _matmul0(x, W0, b0r)
    h0 = _scale0(m0, dego)
    acc0 = _aggregate(h0, srcC, dstC, zblk)
    h1 = _dense1(acc0, degi, dego, g0r, lb0r, W1, b1r)
    acc1 = _aggregate(h1, srcC, dstC, zblk)
    return _dense2(acc1, degi, g1r, lb1r)
